# baseline (speedup 1.0000x reference)
import functools

import jax
import jax.numpy as jnp
from jax import lax
from jax.experimental import pallas as pl
from jax.experimental.pallas import tpu as pltpu
from jax.experimental.pallas import tpu_sc as plsc

N, C, H, W = 5000, 64, 14, 14
OUT = 7
S = 2 * OUT
FHW = H * W
FSZ = C * FHW
OSZ = C * OUT * OUT

NWORK = 32
RPW = 158
BBOX_PAD = NWORK * RPW


_GATHER_DNUMS = lax.GatherDimensionNumbers(
    offset_dims=(), collapsed_slice_dims=(0,), start_index_map=(0,))


def _take(v, idx):
    return lax.gather(v, idx[:, None], _GATHER_DNUMS, slice_sizes=(1,),
                      mode=lax.GatherScatterMode.PROMISE_IN_BOUNDS)


def _roi_align_sc(feature2d, bbox_pad):
    mesh = plsc.VectorSubcoreMesh(core_axis_name="c", subcore_axis_name="s")

    @functools.partial(
        pl.kernel,
        mesh=mesh,
        out_type=jax.ShapeDtypeStruct((N * OSZ,), jnp.float32),
        scratch_types=[
            pltpu.VMEM((RPW * 4 + 16,), jnp.float32),
            pltpu.VMEM((FSZ + 16,), jnp.float32),
            pltpu.VMEM((OSZ + 16,), jnp.float32),
        ],
    )
    def k(feat_hbm, bbox_hbm, out_hbm, bbox_v, fbuf, obuf):
        wid = lax.axis_index("s") * 2 + lax.axis_index("c")
        base = wid * RPW
        pltpu.sync_copy(bbox_hbm.at[pl.ds(base * 4, RPW * 4)],
                        bbox_v.at[pl.ds(0, RPW * 4)])

        lanei = lax.iota(jnp.int32, 16)
        lanef = lanei.astype(jnp.float32)
        mask7 = lanei < OUT
        evenidx = jnp.where(lanei < OUT, lanei * 2, 0)
        oddidx = evenidx + 1
        frac = (lanef + 0.5) * (1.0 / S)

        def roi_body(i, carry):
            n = base + i

            @pl.when(n < N)
            def _():
                pltpu.sync_copy(feat_hbm.at[pl.ds(n * FSZ, FSZ)],
                                fbuf.at[pl.ds(0, FSZ)])
                bbv = bbox_v[pl.ds(i * 4, 16)]
                x1 = bbv[0]
                y1 = bbv[1]
                x2 = bbv[2]
                y2 = bbv[3]
                xs = x1 + frac * (x2 - x1)
                x0i = xs.astype(jnp.int32)
                fx = xs - x0i.astype(jnp.float32)
                x1i = x0i + 1
                ysv = y1 + frac * (y2 - y1)
                y0iv = ysv.astype(jnp.int32)
                fyv = ysv - y0iv.astype(jnp.float32)
                ys_scal = [(y0iv[sy], fyv[sy]) for sy in range(S)]

                def cbody(c, carry2):
                    cbase = c * FHW
                    for oy in range(OUT):
                        y0a, fya = ys_scal[2 * oy]
                        y0b, fyb = ys_scal[2 * oy + 1]
                        a0 = fbuf[pl.ds(cbase + y0a * W, 16)]
                        a1 = fbuf[pl.ds(cbase + y0a * W + W, 16)]
                        ra = a0 + fya * (a1 - a0)
                        b0 = fbuf[pl.ds(cbase + y0b * W, 16)]
                        b1 = fbuf[pl.ds(cbase + y0b * W + W, 16)]
                        rb = b0 + fyb * (b1 - b0)
                        ga0 = _take(ra, x0i)
                        ga1 = _take(ra, x1i)
                        va = ga0 + fx * (ga1 - ga0)
                        gb0 = _take(rb, x0i)
                        gb1 = _take(rb, x1i)
                        vb = gb0 + fx * (gb1 - gb0)
                        m = jnp.maximum(va, vb)
                        o = jnp.maximum(_take(m, evenidx), _take(m, oddidx))
                        obuf[pl.ds(c * (OUT * OUT) + oy * OUT, 16)] = o
                    return carry2

                lax.fori_loop(0, C, cbody, 0)
                pltpu.sync_copy(obuf.at[pl.ds(0, OSZ)],
                                out_hbm.at[pl.ds(n * OSZ, OSZ)])

            return carry

        lax.fori_loop(0, RPW, roi_body, 0)

    return k(feature2d, bbox_pad)


def kernel(feature, bbox):
    feature2d = feature.reshape(N * FSZ)
    bbox_pad = jnp.pad(bbox, ((0, BBOX_PAD - N), (0, 0))).reshape(-1)
    out = _roi_align_sc(feature2d, bbox_pad)
    return out.reshape(N, C, OUT, OUT)


# speedup vs baseline: 1.4170x; 1.4170x over previous
import functools

import jax
import jax.numpy as jnp
from jax import lax
from jax.experimental import pallas as pl
from jax.experimental.pallas import tpu as pltpu
from jax.experimental.pallas import tpu_sc as plsc

N, C, H, W = 5000, 64, 14, 14
OUT = 7
S = 2 * OUT
FHW = H * W
FSZ = C * FHW
OSZ = C * 64
FPAD = FSZ + 16
OPAD = OSZ

NWORK = 32
RPW = 158
NPAIR = RPW // 2
BBOX_PAD = NWORK * RPW

_GATHER_DNUMS = lax.GatherDimensionNumbers(
    offset_dims=(), collapsed_slice_dims=(0,), start_index_map=(0,))


def _take(v, idx):
    return lax.gather(v, idx[:, None], _GATHER_DNUMS, slice_sizes=(1,),
                      mode=lax.GatherScatterMode.PROMISE_IN_BOUNDS)


def _roi_align_sc(feature_flat, bbox_flat):
    mesh = plsc.VectorSubcoreMesh(core_axis_name="c", subcore_axis_name="s")

    @functools.partial(
        pl.kernel,
        mesh=mesh,
        out_type=jax.ShapeDtypeStruct((N * OSZ,), jnp.float32),
        scratch_types=[
            pltpu.VMEM((RPW * 4 + 16,), jnp.float32),
            pltpu.VMEM((FPAD,), jnp.float32),
            pltpu.VMEM((FPAD,), jnp.float32),
            pltpu.VMEM((OPAD,), jnp.float32),
            pltpu.VMEM((OPAD,), jnp.float32),
            pltpu.SemaphoreType.DMA,
            pltpu.SemaphoreType.DMA,
            pltpu.SemaphoreType.DMA,
            pltpu.SemaphoreType.DMA,
        ],
    )
    def k(feat_hbm, bbox_hbm, out_hbm, bbox_v,
          fbuf_a, fbuf_b, obuf_a, obuf_b,
          sem_ia, sem_ib, sem_oa, sem_ob):
        wid = lax.axis_index("s") * 2 + lax.axis_index("c")
        base = wid * RPW
        pltpu.sync_copy(bbox_hbm.at[pl.ds(base * 4, RPW * 4)],
                        bbox_v.at[pl.ds(0, RPW * 4)])

        lanei = lax.iota(jnp.int32, 16)
        lanef = lanei.astype(jnp.float32)
        lane7 = lanei < OUT
        porder = jnp.where(lane7, 2 * lanei,
                           jnp.where(lanei < S, 2 * lanei - 13, 0))
        shiftidx = jnp.minimum(lanei + OUT, 15)
        combidx = jnp.maximum(lanei - OUT, 0)
        frac = (lanef + 0.5) * (1.0 / S)

        def start_in(i, fbuf, sem):
            n = base + i

            @pl.when(n < N)
            def _():
                pltpu.async_copy(feat_hbm.at[pl.ds(n * FSZ, FSZ)],
                                 fbuf.at[pl.ds(0, FSZ)], sem)

        def process(i, fbuf, obuf, sem_i, sem_o):
            n = base + i

            @pl.when(n < N)
            def _():
                pltpu.make_async_copy(feat_hbm.at[pl.ds(0, FSZ)],
                                      fbuf.at[pl.ds(0, FSZ)], sem_i).wait()
                bbv = bbox_v[pl.ds(i * 4, 16)]
                x1 = bbv[0]
                y1 = bbv[1]
                x2 = bbv[2]
                y2 = bbv[3]
                xs = x1 + frac * (x2 - x1)
                x0i = xs.astype(jnp.int32)
                fx = xs - x0i.astype(jnp.float32)
                xp0 = _take(x0i, porder)
                xp1 = xp0 + 1
                fxp = _take(fx, porder)
                ysv = y1 + frac * (y2 - y1)
                y0iv = ysv.astype(jnp.int32)
                fyv = ysv - y0iv.astype(jnp.float32)
                ys_scal = [(y0iv[sy], fyv[sy]) for sy in range(S)]

                @plsc.parallel_loop(0, C, unroll=2)
                def cbody(c):
                    cbase = c * FHW
                    cob = c * 64
                    os = []
                    for oy in range(OUT):
                        y0a, fya = ys_scal[2 * oy]
                        y0b, fyb = ys_scal[2 * oy + 1]
                        a0 = fbuf[pl.ds(cbase + y0a * W, 16)]
                        a1 = fbuf[pl.ds(cbase + y0a * W + W, 16)]
                        ra = a0 + fya * (a1 - a0)
                        b0 = fbuf[pl.ds(cbase + y0b * W, 16)]
                        b1 = fbuf[pl.ds(cbase + y0b * W + W, 16)]
                        rb = b0 + fyb * (b1 - b0)
                        ga0 = _take(ra, xp0)
                        ga1 = _take(ra, xp1)
                        va = ga0 + fxp * (ga1 - ga0)
                        gb0 = _take(rb, xp0)
                        gb1 = _take(rb, xp1)
                        vb = gb0 + fxp * (gb1 - gb0)
                        m = jnp.maximum(va, vb)
                        os.append(jnp.maximum(m, _take(m, shiftidx)))
                    for p in range(3):
                        comb = jnp.where(lane7, os[2 * p],
                                         _take(os[2 * p + 1], combidx))
                        obuf[pl.ds(cob + p * 16, 16)] = comb
                    obuf[pl.ds(cob + 48, 16)] = os[6]

                pltpu.async_copy(obuf.at[pl.ds(0, OSZ)],
                                 out_hbm.at[pl.ds(n * OSZ, OSZ)], sem_o)

        def wait_out(sem):
            pltpu.make_async_copy(obuf_a.at[pl.ds(0, OSZ)],
                                  out_hbm.at[pl.ds(0, OSZ)], sem).wait()

        start_in(0, fbuf_a, sem_ia)

        def pair_body(p, carry):
            i0 = p * 2
            start_in(i0 + 1, fbuf_b, sem_ib)

            @pl.when((i0 >= 2) & (base + i0 - 2 < N))
            def _():
                wait_out(sem_oa)

            process(i0, fbuf_a, obuf_a, sem_ia, sem_oa)

            @pl.when(p < NPAIR - 1)
            def _():
                start_in(i0 + 2, fbuf_a, sem_ia)

            @pl.when((i0 >= 2) & (base + i0 - 1 < N))
            def _():
                wait_out(sem_ob)

            process(i0 + 1, fbuf_b, obuf_b, sem_ib, sem_ob)
            return carry

        lax.fori_loop(0, NPAIR, pair_body, 0)
        @pl.when(base + (RPW - 2) < N)
        def _():
            wait_out(sem_oa)

        @pl.when(base + (RPW - 1) < N)
        def _():
            wait_out(sem_ob)

    return k(feature_flat, bbox_flat)


def kernel(feature, bbox):
    feature_flat = feature.reshape(N * FSZ)
    bbox_pad = jnp.pad(bbox, ((0, BBOX_PAD - N), (0, 0))).reshape(-1)
    out = _roi_align_sc(feature_flat, bbox_pad)
    out = out.reshape(N, C, 4, 16)[:, :, :, :S]
    out = out.reshape(N, C, 8, OUT)[:, :, :OUT, :]
    return out
